# baseline (speedup 1.0000x reference)
import jax
import jax.numpy as jnp
from jax.experimental import pallas as pl
from jax.experimental.pallas import tpu as pltpu


def _mxu_probe(x_ref, o_ref):
    a = x_ref[...].astype(jnp.bfloat16)
    w = a[0:128, :]

    def body(k, acc):
        return acc + jnp.dot(a, w, preferred_element_type=jnp.float32)

    acc = jax.lax.fori_loop(0, 1000, body, jnp.zeros((1024, 128), jnp.float32))
    o_ref[...] = acc[0:8, :]


def kernel(x, *rest):
    xs = x.reshape(32 * 256, 32 * 32)[0:1024, 0:128]
    out = pl.pallas_call(
        _mxu_probe,
        out_shape=jax.ShapeDtypeStruct((8, 128), jnp.float32),
        grid=(1,),
        in_specs=[pl.BlockSpec((1024, 128), lambda b: (0, 0))],
        out_specs=pl.BlockSpec((8, 128), lambda b: (0, 0)),
    )(xs)
    return jnp.zeros((32, 256, 32, 32), jnp.float32) + out[0, 0]


# speedup vs baseline: 1.5912x; 1.5912x over previous
import functools

import jax
import jax.numpy as jnp
from jax.experimental import pallas as pl
from jax.experimental.pallas import tpu as pltpu

_EPS = 1e-5
_BF16 = jnp.bfloat16
_TA = (((0,), (0,)), ((), ()))


def _silu(y2):
    return y2 + y2 * jnp.tanh(y2)


def _dot(a, b):
    return jnp.dot(a, b, preferred_element_type=jnp.float32)


def _csp_kernel(w_sp, pad, n_img,
                x_ref, wx_ref, wsq_ref, wk_ref, w4_ref, bs_ref, b4_ref,
                o_ref, sl_ref, sm_ref, sr_ref):
    hw = o_ref.shape[2]
    c = wsq_ref.shape[2]

    zb = jnp.zeros((1, w_sp + 1, c), _BF16)
    for s in (sl_ref, sm_ref, sr_ref):
        for i in range(n_img):
            s[i:i + 1, pad - w_sp:pad + 1, :] = zb
            s[i:i + 1, pad + hw - 1:pad + hw + w_sp, :] = zb

    ri = jax.lax.broadcasted_iota(jnp.int32, (hw, 1), 0) % w_sp
    mL = ri != 0
    mR = ri != w_sp - 1
    zero = jnp.zeros((), _BF16)

    hs = []
    a2s = []
    for i in range(n_img):
        xb = x_ref[i].astype(_BF16)
        hs.append(_silu(jax.lax.dot_general(xb, wx_ref[0], _TA,
                                            preferred_element_type=jnp.float32)
                        + bs_ref[0:1]))
        a2s.append(_silu(jax.lax.dot_general(xb, wx_ref[1], _TA,
                                             preferred_element_type=jnp.float32)
                         + bs_ref[8:9]).astype(_BF16))

    for blk in range(3):
        wc1 = wsq_ref[blk]
        bc1 = bs_ref[1 + 2 * blk:2 + 2 * blk]
        bc2 = bs_ref[2 + 2 * blk:3 + 2 * blk]
        tls, tms, trs = [], [], []
        for i in range(n_img):
            t = _silu(_dot(hs[i].astype(_BF16), wc1) + bc1)
            tb = t.astype(_BF16)
            tl = jnp.where(mL, tb, zero)
            tr = jnp.where(mR, tb, zero)
            sl_ref[i, pad - 1:pad - 1 + hw, :] = tl
            sm_ref[i, pad:pad + hw, :] = tb
            sr_ref[i, pad + 1:pad + 1 + hw, :] = tr
            tls.append(tl); tms.append(tb); trs.append(tr)
        for i in range(n_img):
            acc = None
            for kh in range(3):
                base = pad + (kh - 1) * w_sp
                for j, s, v in ((0, sl_ref, tls[i]), (1, sm_ref, tms[i]),
                                (2, sr_ref, trs[i])):
                    op = v if (kh == 1 and j == 1) else s[i, base:base + hw, :]
                    d = _dot(op, wk_ref[blk, kh, j * c:(j + 1) * c])
                    acc = d if acc is None else acc + d
            hs[i] = _silu(acc + bc2) + hs[i]

    for i in range(n_img):
        a1 = _silu(_dot(hs[i].astype(_BF16), wsq_ref[3]) + bs_ref[7:8])
        z = (_dot(a1.astype(_BF16), w4_ref[0])
             + _dot(a2s[i], w4_ref[1])
             + b4_ref[...])
        o_ref[i] = _silu(z).T.astype(o_ref.dtype)


def _fold(w2d, gamma, beta, mean, var):
    s = 0.5 * gamma / jnp.sqrt(var + _EPS)
    return w2d * s[:, None], 0.5 * beta - mean * s


def kernel(x, cv1_w, cv1_bn_g, cv1_bn_b, cv1_bn_m, cv1_bn_v, cv2_w, cv3_w,
           cv4_w, cv4_bn_g, cv4_bn_b, cv4_bn_m, cv4_bn_v,
           bn_g, bn_b, bn_m, bn_v,
           m0_c1_w, m0_c1_bn_g, m0_c1_bn_b, m0_c1_bn_m, m0_c1_bn_v,
           m0_c2_w, m0_c2_bn_g, m0_c2_bn_b, m0_c2_bn_m, m0_c2_bn_v,
           m1_c1_w, m1_c1_bn_g, m1_c1_bn_b, m1_c1_bn_m, m1_c1_bn_v,
           m1_c2_w, m1_c2_bn_g, m1_c2_bn_b, m1_c2_bn_m, m1_c2_bn_v,
           m2_c1_w, m2_c1_bn_g, m2_c1_bn_b, m2_c1_bn_m, m2_c1_bn_v,
           m2_c2_w, m2_c2_bn_g, m2_c2_bn_b, m2_c2_bn_m, m2_c2_bn_v):
    B, c1, H, W = x.shape
    hw = H * W
    c_ = cv1_w.shape[0]
    c2 = cv4_w.shape[0]
    xf = x.reshape(B, c1, hw)
    pad = 2 * W

    w1, b1 = _fold(cv1_w[:, :, 0, 0], cv1_bn_g, cv1_bn_b, cv1_bn_m, cv1_bn_v)
    sa = 0.5 * bn_g[:c_] / jnp.sqrt(bn_v[:c_] + _EPS)
    ta = 0.5 * bn_b[:c_] - bn_m[:c_] * sa
    sb = 0.5 * bn_g[c_:] / jnp.sqrt(bn_v[c_:] + _EPS)
    tb = 0.5 * bn_b[c_:] - bn_m[c_:] * sb
    w3f = cv3_w[:, :, 0, 0] * sa[:, None]
    w2f = cv2_w[:, :, 0, 0] * sb[:, None]
    w4f, b4f = _fold(cv4_w[:, :, 0, 0], cv4_bn_g, cv4_bn_b, cv4_bn_m,
                     cv4_bn_v)

    wsq = [None, None, None, w3f.T]
    wks = []
    biases = [b1, None, None, None, None, None, None, ta, tb]
    for blk, (c1w, g1, bb1, mm1, v1, c2w, g2, bb2, mm2, v2) in enumerate((
            (m0_c1_w, m0_c1_bn_g, m0_c1_bn_b, m0_c1_bn_m, m0_c1_bn_v,
             m0_c2_w, m0_c2_bn_g, m0_c2_bn_b, m0_c2_bn_m, m0_c2_bn_v),
            (m1_c1_w, m1_c1_bn_g, m1_c1_bn_b, m1_c1_bn_m, m1_c1_bn_v,
             m1_c2_w, m1_c2_bn_g, m1_c2_bn_b, m1_c2_bn_m, m1_c2_bn_v),
            (m2_c1_w, m2_c1_bn_g, m2_c1_bn_b, m2_c1_bn_m, m2_c1_bn_v,
             m2_c2_w, m2_c2_bn_g, m2_c2_bn_b, m2_c2_bn_m, m2_c2_bn_v))):
        wi, bi = _fold(c1w[:, :, 0, 0], g1, bb1, mm1, v1)
        wsq[blk] = wi.T
        s2 = 0.5 * g2 / jnp.sqrt(v2 + _EPS)
        wc = jnp.transpose(c2w * s2[:, None, None, None], (2, 3, 1, 0))
        wks.append(wc[:, ::-1].reshape(3, 3 * c_, c_))
        biases[1 + 2 * blk] = bi
        biases[2 + 2 * blk] = 0.5 * bb2 - mm2 * s2

    wx = jnp.stack([w1.T, w2f.T]).astype(_BF16)
    wsq = jnp.stack(wsq).astype(_BF16)
    wk = jnp.stack(wks).astype(_BF16)
    w4 = jnp.stack([w4f[:, :c_].T, w4f[:, c_:].T]).astype(_BF16)
    bs = jnp.stack(biases)
    args = [xf, wx, wsq, wk, w4, bs, b4f[None, :]]

    def full(a):
        return pl.BlockSpec(a.shape, lambda bi: (0,) * a.ndim)

    n_img = 4
    in_specs = [pl.BlockSpec((n_img, c1, hw), lambda bi: (bi, 0, 0))]
    in_specs += [full(a) for a in args[1:]]

    out = pl.pallas_call(
        functools.partial(_csp_kernel, W, pad, n_img),
        out_shape=jax.ShapeDtypeStruct((B, c2, hw), x.dtype),
        grid=(B // n_img,),
        in_specs=in_specs,
        out_specs=pl.BlockSpec((n_img, c2, hw), lambda bi: (bi, 0, 0)),
        scratch_shapes=[pltpu.VMEM((n_img, hw + 2 * pad, c_), _BF16)] * 3,
        compiler_params=pltpu.CompilerParams(
            dimension_semantics=("parallel",)),
    )(*args)
    return out.reshape(B, c2, H, W)
